# baseline (speedup 1.0000x reference)
import functools

import numpy as np

import jax
import jax.numpy as jnp
from jax import lax
from jax.experimental import pallas as pl
from jax.experimental.pallas import tpu as pltpu
from jax.experimental.pallas import tpu_sc as plsc

B = 4
C = 128
BC = B * C
IN_V = 16000
VP = 16384
DEG = 32
V2 = VP // 4
NC = 2
NS = 16
NW = NC * NS
NG = BC // 32
_ILEAVE = plsc.PackFormat.INTERLEAVED


def _pack_perm(d):
    i16 = np.arange(16)
    il = np.stack([i16, i16 + 16], axis=1).reshape(32)
    return (np.arange(d // 32)[:, None] * 32 + il[None, :]).reshape(d)


def _worker_id():
    return lax.axis_index("s") * NC + lax.axis_index("c")


def _sc_perm_gather(src, perm):
    V = perm.shape[0]
    W32 = BC // 2
    rpw = V // NW
    CH = 64

    mesh = plsc.VectorSubcoreMesh(core_axis_name="c", subcore_axis_name="s")

    @functools.partial(
        pl.kernel,
        mesh=mesh,
        out_type=jax.ShapeDtypeStruct((V, W32), jnp.int32),
        scratch_types=[
            pltpu.VMEM((CH,), jnp.int32),
            pltpu.VMEM((CH, W32), jnp.int32),
            pltpu.SemaphoreType.DMA,
        ],
    )
    def k(src_hbm, perm_hbm, out_hbm, idx_v, rows_v, sem):
        wid = _worker_id()

        def chunk(t, carry):
            base = wid * rpw + t * CH
            pltpu.sync_copy(perm_hbm.at[pl.ds(base, CH)], idx_v)
            pltpu.async_copy(src_hbm.at[idx_v], rows_v, sem).wait()
            pltpu.sync_copy(rows_v, out_hbm.at[pl.ds(base, CH)])
            return carry

        lax.fori_loop(0, rpw // CH, chunk, 0)

    return k(src, perm)


def _sc_spmm(m, cols, vals):
    V = m.shape[0]
    W32 = BC // 2
    rpw = V // NW
    cols2 = cols.reshape(V, DEG)
    vals2 = vals.reshape(V, DEG)

    mesh = plsc.VectorSubcoreMesh(core_axis_name="c", subcore_axis_name="s")

    CHI = 64
    H = 16

    @functools.partial(
        pl.kernel,
        mesh=mesh,
        out_type=jax.ShapeDtypeStruct((V, W32), jnp.int32),
        scratch_types=[
            pltpu.VMEM((CHI, DEG), jnp.int32),
            pltpu.VMEM((CHI, DEG), jnp.float32),
            pltpu.VMEM((2, H, W32), jnp.int32),
            pltpu.VMEM((W32,), jnp.int32),
            pltpu.SemaphoreType.DMA,
            pltpu.SemaphoreType.DMA,
        ],
    )
    def k(m_hbm, cols_hbm, vals_hbm, out_hbm, idxc, valsc, rows2, stage_v, sem0, sem1):
        wid = _worker_id()
        r0g = wid * rpw

        def compute_half(q, p, buf, acc):
            vsrc = valsc[q, pl.ds(p * H, 16)]
            himask = jnp.full((16,), jnp.int32(-65536))

            def edge(j, a):
                vj = vsrc.at[jnp.full((16,), j, jnp.int32)].get(
                    mode="promise_in_bounds")
                new = []
                for g in range(NG):
                    w = rows2[buf, j, pl.ds(16 * g, 16)]
                    lo = lax.bitcast_convert_type(w << 16, jnp.float32)
                    hi = lax.bitcast_convert_type(w & himask, jnp.float32)
                    new.append(a[2 * g] + vj * lo)
                    new.append(a[2 * g + 1] + vj * hi)
                return tuple(new)

            return lax.fori_loop(0, H, edge, acc)

        def pack_bf16(a_f32, b_f32):
            one = jnp.full((16,), jnp.int32(1))
            rnd = jnp.full((16,), jnp.int32(0x7FFF))
            ba = lax.bitcast_convert_type(a_f32, jnp.int32)
            bb = lax.bitcast_convert_type(b_f32, jnp.int32)
            ra = lax.shift_right_logical(
                ba + rnd + (lax.shift_right_logical(ba, 16) & one), 16)
            rb = lax.shift_right_logical(
                bb + rnd + (lax.shift_right_logical(bb, 16) & one), 16)
            return ra | (rb << 16)

        def blk_body(blk, carry):
            row0 = r0g + blk * CHI
            pltpu.sync_copy(cols_hbm.at[pl.ds(row0, CHI)], idxc)
            pltpu.sync_copy(vals_hbm.at[pl.ds(row0, CHI)], valsc)
            pltpu.async_copy(m_hbm.at[idxc.at[0, pl.ds(0, H)]], rows2.at[0], sem0)

            def row_body(q, c2):
                pltpu.async_copy(
                    m_hbm.at[idxc.at[q, pl.ds(H, H)]], rows2.at[1], sem1)
                pltpu.make_async_copy(
                    m_hbm.at[idxc.at[q, pl.ds(0, H)]], rows2.at[0], sem0).wait()
                acc0 = tuple(jnp.zeros((16,), jnp.float32)
                             for _ in range(2 * NG))
                acc = compute_half(q, 0, 0, acc0)

                @pl.when(q + 1 < CHI)
                def _prefetch():
                    pltpu.async_copy(
                        m_hbm.at[idxc.at[q + 1, pl.ds(0, H)]], rows2.at[0], sem0)

                pltpu.make_async_copy(
                    m_hbm.at[idxc.at[q, pl.ds(H, H)]], rows2.at[1], sem1).wait()
                acc = compute_half(q, 1, 1, acc)
                for g in range(NG):
                    stage_v[pl.ds(16 * g, 16)] = pack_bf16(
                        acc[2 * g], acc[2 * g + 1])
                pltpu.sync_copy(stage_v, out_hbm.at[row0 + q])
                return c2

            lax.fori_loop(0, CHI, row_body, 0)
            return carry

        lax.fori_loop(0, rpw // CHI, blk_body, 0)

    return k(m, cols2, vals2)


def _tc_conv(xs0, xs1, xst, w, bias, vt, out_dtype):
    V = xs0.shape[0]
    F = w.shape[0]

    def body(x0_ref, x1_ref, xt_ref, w_ref, b_ref, o_ref):
        for b in range(B):
            sl = pl.ds(b * C, C)
            X = jnp.concatenate(
                [x0_ref[:, sl], x1_ref[:, sl], xt_ref[:, sl]],
                axis=1).astype(jnp.float32)
            Y = lax.dot_general(X, w_ref[...], (((1,), (1,)), ((), ())),
                                preferred_element_type=jnp.float32)
            Z = jnp.maximum(Y + b_ref[...], 0.0)
            o_ref[:, b, :] = Z.reshape(vt // 4, 4, F).max(axis=1).astype(out_dtype)

    grid = (V // vt,)
    xspec = pl.BlockSpec((vt, BC), lambda i: (i, 0))
    return pl.pallas_call(
        body,
        grid=grid,
        in_specs=[
            xspec, xspec, xspec,
            pl.BlockSpec((F, 3 * C), lambda i: (0, 0)),
            pl.BlockSpec((1, F), lambda i: (0, 0)),
        ],
        out_specs=pl.BlockSpec((vt // 4, B, F), lambda i: (i, 0, 0)),
        out_shape=jax.ShapeDtypeStruct((V // 4, B, F), out_dtype),
    )(xs0, xs1, xst, w, bias)


def _tc_fc(act, wfc, bias, kt):
    Bx, K = act.shape
    F = wfc.shape[0]

    def body(a_ref, w_ref, b_ref, o_ref):
        @pl.when(pl.program_id(0) == 0)
        def _init():
            o_ref[...] = jnp.broadcast_to(b_ref[...], o_ref.shape)

        o_ref[...] += lax.dot_general(
            a_ref[...], w_ref[...], (((1,), (1,)), ((), ())),
            preferred_element_type=jnp.float32)

    return pl.pallas_call(
        body,
        grid=(K // kt,),
        in_specs=[
            pl.BlockSpec((Bx, kt), lambda k: (0, k)),
            pl.BlockSpec((F, kt), lambda k: (0, k)),
            pl.BlockSpec((1, F), lambda k: (0, 0)),
        ],
        out_specs=pl.BlockSpec((Bx, F), lambda k: (0, 0)),
        out_shape=jax.ShapeDtypeStruct((Bx, F), jnp.float32),
    )(act, wfc, bias)


def _prep_w(W, scale):
    F = W.shape[0]
    Wk = W.reshape(F, C, 3)
    W0 = Wk[:, :, 0] - Wk[:, :, 2]
    Wa = Wk[:, :, 1]
    Wb = 2.0 * Wk[:, :, 2]
    We = jnp.concatenate([W0, Wa, Wb], axis=1) * scale
    return We[:, _pack_perm(3 * C)]


def kernel(x, perm, L1_rows, L1_cols, L1_vals, L2_rows, L2_cols, L2_vals,
           W1, b1, W2, b2, Wfc1, bfc1):
    del L1_rows, L2_rows
    s = 1.0 / jnp.sqrt(jnp.float32(1.0 + 1e-5))

    xr = jnp.transpose(x, (2, 0, 1)).reshape(IN_V, BC)
    xr = jnp.pad(xr, ((0, VP - IN_V), (0, 0)))
    xr = xr.reshape(VP, NG, 2, 16).swapaxes(2, 3).reshape(VP, BC)
    xr = xr.astype(jnp.bfloat16)
    xr32 = lax.bitcast_convert_type(xr.reshape(VP, BC // 2, 2), jnp.int32)

    def tobf(a):
        return lax.bitcast_convert_type(a, jnp.bfloat16).reshape(
            a.shape[0], BC)

    x0 = _sc_perm_gather(xr32, perm)
    x1 = _sc_spmm(x0, L1_cols, L1_vals)
    t1 = _sc_spmm(x1, L1_cols, L1_vals)
    w1e = _prep_w(W1, s)
    outp = _pack_perm(C)
    p1 = _tc_conv(tobf(x0), tobf(x1), tobf(t1),
                  w1e[outp, :], b1[outp].reshape(1, -1),
                  vt=1024, out_dtype=jnp.bfloat16)
    m2bf = p1.reshape(V2, BC)
    m2 = lax.bitcast_convert_type(m2bf.reshape(V2, BC // 2, 2), jnp.int32)

    x1b = _sc_spmm(m2, L2_cols, L2_vals)
    t2 = _sc_spmm(x1b, L2_cols, L2_vals)
    w2e = _prep_w(W2, 1.0)
    p2 = _tc_conv(m2bf, tobf(x1b), tobf(t2), w2e, b2.reshape(1, -1),
                  vt=1024, out_dtype=jnp.float32)

    act = jnp.transpose(p2, (1, 2, 0)).reshape(B, -1)
    return _tc_fc(act, Wfc1, bfc1.reshape(1, -1), kt=8192)
